# baseline (speedup 1.0000x reference)
import functools

import jax
import jax.numpy as jnp
from jax import lax
from jax.experimental import pallas as pl
from jax.experimental.pallas import tpu as pltpu
from jax.experimental.pallas import tpu_sc as plsc

N = 10000
E = 320000
D = 128
D_EMB = 64
D_OUT = 40
EPS_BN = 1e-5

NC = 2
NS = 16
NPAD = 10240
HALF = NPAD // NC
HALFP = 5248
TRASH = HALF
CH = 128
EPT = E // NS
KCH = 160
EPT_PAD = KCH * CH
RPT = HALFP // NS


def _deg_body(dst_hbm, ones_hbm, zeros_hbm, out_hbm, di_v, ones_v, deg_sh):
    c = lax.axis_index("c")
    s = lax.axis_index("s")
    wid = c * NS + s
    row0 = s * RPT
    pltpu.sync_copy(dst_hbm.at[wid], di_v)
    pltpu.sync_copy(ones_hbm, ones_v)
    pltpu.sync_copy(zeros_hbm.at[pl.ds(row0, RPT)],
                    deg_sh.at[pl.ds(row0, RPT)])
    plsc.subcore_barrier()

    def chunk_body(j, carry):
        pltpu.sync_copy(ones_v, deg_sh.at[di_v.at[j]], add=True)
        return carry

    lax.fori_loop(0, KCH, chunk_body, 0)
    plsc.subcore_barrier()
    pltpu.sync_copy(deg_sh.at[pl.ds(row0, RPT)],
                    out_hbm.at[c].at[pl.ds(row0, RPT)])


NB = 2


def _agg_body(z_hbm, src_hbm, dst_hbm, zeros_hbm, p_hbm,
              si_v, di_v, gbuf0, gbuf1,
              semg0, semg1, sems0, sems1, agg_sh):
    c = lax.axis_index("c")
    s = lax.axis_index("s")
    wid = c * NS + s
    row0 = s * RPT
    pltpu.sync_copy(src_hbm.at[s], si_v)
    pltpu.sync_copy(dst_hbm.at[wid], di_v)
    pltpu.sync_copy(zeros_hbm.at[pl.ds(row0, RPT)],
                    agg_sh.at[pl.ds(row0, RPT)])
    plsc.subcore_barrier()

    gbufs = (gbuf0, gbuf1)
    semg = (semg0, semg1)
    sems = (sems0, sems1)
    for b in range(NB):
        pltpu.async_copy(z_hbm.at[si_v.at[b]], gbufs[b], semg[b])

    def ring_body(i, carry):
        j0 = i * NB
        for b in range(NB):
            j = j0 + b
            pltpu.make_async_copy(z_hbm.at[si_v.at[0]], gbufs[b],
                                  semg[b]).wait()
            pltpu.async_copy(gbufs[b], agg_sh.at[di_v.at[j]], sems[b],
                             add=True)
        for b in range(NB):
            j = j0 + b

            @pl.when(j + NB < KCH)
            def _():
                pltpu.make_async_copy(gbufs[b], agg_sh.at[di_v.at[0]],
                                      sems[b]).wait()
                pltpu.async_copy(z_hbm.at[si_v.at[j + NB]], gbufs[b],
                                 semg[b])
        return carry

    lax.fori_loop(0, KCH // NB, ring_body, 0)
    for b in range(NB):
        pltpu.make_async_copy(gbufs[b], agg_sh.at[di_v.at[0]], sems[b]).wait()
    plsc.subcore_barrier()
    pltpu.sync_copy(agg_sh.at[pl.ds(row0, RPT)],
                    p_hbm.at[c].at[pl.ds(row0, RPT)])


@functools.cache
def _sc_kernels():
    mesh = plsc.VectorSubcoreMesh(core_axis_name="c", subcore_axis_name="s")
    deg = pl.kernel(
        _deg_body,
        out_type=jax.ShapeDtypeStruct((NC, HALFP, D), jnp.float32),
        mesh=mesh,
        scratch_types=[
            pltpu.VMEM((KCH, CH), jnp.int32),
            pltpu.VMEM((CH, D), jnp.float32),
            pltpu.VMEM_SHARED((HALFP, D), jnp.float32),
        ],
    )
    agg = pl.kernel(
        _agg_body,
        out_type=jax.ShapeDtypeStruct((NC, HALFP, D), jnp.float32),
        mesh=mesh,
        scratch_types=(
            [pltpu.VMEM((KCH, CH), jnp.int32),
             pltpu.VMEM((KCH, CH), jnp.int32)]
            + [pltpu.VMEM((CH, D), jnp.float32)] * NB
            + [pltpu.SemaphoreType.DMA] * (2 * NB)
            + [pltpu.VMEM_SHARED((HALFP, D), jnp.float32)]
        ),
    )
    return deg, agg


def _tc_a_body(hist_ref, x_ref, w_ref, dinv_ref, z_ref):
    counts = jnp.concatenate(
        [hist_ref[0, :HALF, :1], hist_ref[1, :HALF, :1]], axis=0)
    deg = counts + 1.0
    row = lax.broadcasted_iota(jnp.int32, (NPAD, 1), 0)
    dinv = jnp.where(row < N, lax.rsqrt(deg), 0.0)
    dinv_ref[...] = dinv
    z = jnp.dot(x_ref[...], w_ref[...], preferred_element_type=jnp.float32)
    z_ref[...] = z * dinv


_tc_a = pl.pallas_call(
    _tc_a_body,
    out_shape=(
        jax.ShapeDtypeStruct((NPAD, 1), jnp.float32),
        jax.ShapeDtypeStruct((NPAD, D), jnp.float32),
    ),
)


def _combine_bn_relu(p_ref, z_ref, dinv_ref, b_ref, g_ref, be_ref):
    dinv = dinv_ref[...]
    agg = jnp.concatenate([p_ref[0, :HALF], p_ref[1, :HALF]], axis=0)
    agg = agg + z_ref[...]
    pre = agg * dinv + b_ref[...]
    mask = lax.broadcasted_iota(jnp.int32, (NPAD, 1), 0) < N
    pre = jnp.where(mask, pre, 0.0)
    mean = jnp.sum(pre, axis=0, keepdims=True) * (1.0 / N)
    cent = jnp.where(mask, pre - mean, 0.0)
    var = jnp.sum(cent * cent, axis=0, keepdims=True) * (1.0 / N)
    h = g_ref[...] * cent * lax.rsqrt(var + EPS_BN) + be_ref[...]
    return jnp.maximum(h, 0.0)


def _tc_b_body(p_ref, z_ref, dinv_ref, b_ref, g_ref, be_ref, w_ref, z2_ref):
    h = _combine_bn_relu(p_ref, z_ref, dinv_ref, b_ref, g_ref, be_ref)
    z2 = jnp.dot(h, w_ref[...], preferred_element_type=jnp.float32)
    z2_ref[...] = z2 * dinv_ref[...]


_tc_b = pl.pallas_call(
    _tc_b_body,
    out_shape=jax.ShapeDtypeStruct((NPAD, D), jnp.float32),
)


def _tc_c_body(p_ref, z_ref, dinv_ref, b_ref, g_ref, be_ref, fair_ref,
               wl1a_ref, wl1b_ref, bl1_ref, wl2_ref, bl2_ref, out_ref):
    h = _combine_bn_relu(p_ref, z_ref, dinv_ref, b_ref, g_ref, be_ref)
    hl = jnp.dot(h, wl1a_ref[...], preferred_element_type=jnp.float32)
    hl = hl + jnp.dot(fair_ref[...], wl1b_ref[...],
                      preferred_element_type=jnp.float32)
    hl = jnp.maximum(hl + bl1_ref[...], 0.0)
    out = jnp.dot(hl, wl2_ref[...], preferred_element_type=jnp.float32)
    out_ref[...] = out + bl2_ref[...]


_tc_c = pl.pallas_call(
    _tc_c_body,
    out_shape=jax.ShapeDtypeStruct((NPAD, D_OUT), jnp.float32),
)


def kernel(x, adj_t, fair_node_embedding, W1, b1, g1, be1, W2, b2, g2, be2,
           Wl1, bl1, Wl2, bl2):
    src = adj_t[0].reshape(NS, EPT)
    dst = adj_t[1].reshape(NS, EPT)
    pad = jnp.full((NS, EPT_PAD - EPT), N, dtype=adj_t.dtype)
    src_p3 = jnp.concatenate([src, pad], axis=1).reshape(NS, KCH, CH)
    dst_p = jnp.concatenate([dst, pad], axis=1)
    dloc = []
    for c in range(NC):
        d = dst_p - c * HALF
        dloc.append(jnp.where((d >= 0) & (d < HALF), d, TRASH))
    dst_loc = jnp.stack(dloc).reshape(NC * NS, KCH, CH)

    x_pad = jnp.pad(x, ((0, NPAD - N), (0, 0)))
    fair_pad = jnp.pad(fair_node_embedding, ((0, NPAD - N), (0, 0)))
    zeros = jnp.zeros((HALFP, D), jnp.float32)
    ones = jnp.ones((CH, D), jnp.float32)

    deg_kernel, agg_kernel = _sc_kernels()
    hist = deg_kernel(dst_loc, ones, zeros)
    dinv, z1 = _tc_a(hist, x_pad, W1)
    p1 = agg_kernel(z1, src_p3, dst_loc, zeros)
    z2 = _tc_b(p1, z1, dinv, b1.reshape(1, D), g1.reshape(1, D),
               be1.reshape(1, D), W2)
    p2 = agg_kernel(z2, src_p3, dst_loc, zeros)
    out = _tc_c(p2, z2, dinv, b2.reshape(1, D), g2.reshape(1, D),
                be2.reshape(1, D), fair_pad, Wl1[:D], Wl1[D:],
                bl1.reshape(1, D), Wl2, bl2.reshape(1, D_OUT))
    return out[:N]


# speedup vs baseline: 1.0678x; 1.0678x over previous
import functools

import jax
import jax.numpy as jnp
from jax import lax
from jax.experimental import pallas as pl
from jax.experimental.pallas import tpu as pltpu
from jax.experimental.pallas import tpu_sc as plsc

N = 10000
E = 320000
D = 128
D_EMB = 64
D_OUT = 40
EPS_BN = 1e-5

NC = 2
NS = 16
NPAD = 10240
HALF = NPAD // NC
HALFP = 5248
TRASH = HALF
CH = 128
EPT = E // NS
KCH = 160
EPT_PAD = KCH * CH
RPT = HALFP // NS


def _deg_body(dst_hbm, ones_hbm, zeros_hbm, out_hbm, di_v, ones_v, deg_sh):
    c = lax.axis_index("c")
    s = lax.axis_index("s")
    wid = c * NS + s
    row0 = s * RPT
    pltpu.sync_copy(dst_hbm.at[wid], di_v)
    pltpu.sync_copy(ones_hbm, ones_v)
    pltpu.sync_copy(zeros_hbm.at[pl.ds(row0, RPT)],
                    deg_sh.at[pl.ds(row0, RPT)])
    plsc.subcore_barrier()

    def chunk_body(j, carry):
        pltpu.sync_copy(ones_v, deg_sh.at[di_v.at[j]], add=True)
        return carry

    lax.fori_loop(0, KCH, chunk_body, 0)
    plsc.subcore_barrier()
    pltpu.sync_copy(deg_sh.at[pl.ds(row0, RPT)],
                    out_hbm.at[c].at[pl.ds(row0, RPT)])


NB = 2


def _agg_body(z_hbm, src_hbm, dst_hbm, zeros_hbm, p_hbm,
              si_v, di_v, gbuf0, gbuf1,
              semg0, semg1, sems0, sems1, agg_sh):
    c = lax.axis_index("c")
    s = lax.axis_index("s")
    wid = c * NS + s
    row0 = s * RPT
    pltpu.sync_copy(src_hbm.at[s], si_v)
    pltpu.sync_copy(dst_hbm.at[wid], di_v)
    pltpu.sync_copy(zeros_hbm.at[pl.ds(row0, RPT)],
                    agg_sh.at[pl.ds(row0, RPT)])
    plsc.subcore_barrier()

    gbufs = (gbuf0, gbuf1)
    semg = (semg0, semg1)
    sems = (sems0, sems1)
    for b in range(NB):
        pltpu.async_copy(z_hbm.at[si_v.at[b]], gbufs[b], semg[b])

    def ring_body(i, carry):
        j0 = i * NB
        for b in range(NB):
            j = j0 + b
            pltpu.make_async_copy(z_hbm.at[si_v.at[0]], gbufs[b],
                                  semg[b]).wait()
            pltpu.async_copy(gbufs[b], agg_sh.at[di_v.at[j]], sems[b],
                             add=True)
        for b in range(NB):
            j = j0 + b

            @pl.when(j + NB < KCH)
            def _():
                pltpu.make_async_copy(gbufs[b], agg_sh.at[di_v.at[0]],
                                      sems[b]).wait()
                pltpu.async_copy(z_hbm.at[si_v.at[j + NB]], gbufs[b],
                                 semg[b])
        return carry

    lax.fori_loop(0, KCH // NB, ring_body, 0)
    for b in range(NB):
        pltpu.make_async_copy(gbufs[b], agg_sh.at[di_v.at[0]], sems[b]).wait()
    plsc.subcore_barrier()
    pltpu.sync_copy(agg_sh.at[pl.ds(row0, RPT)],
                    p_hbm.at[c].at[pl.ds(row0, RPT)])


@functools.cache
def _sc_kernels():
    mesh = plsc.VectorSubcoreMesh(core_axis_name="c", subcore_axis_name="s")
    deg = pl.kernel(
        _deg_body,
        out_type=jax.ShapeDtypeStruct((NC, HALFP, D), jnp.float32),
        mesh=mesh,
        scratch_types=[
            pltpu.VMEM((KCH, CH), jnp.int32),
            pltpu.VMEM((CH, D), jnp.float32),
            pltpu.VMEM_SHARED((HALFP, D), jnp.float32),
        ],
    )
    agg = pl.kernel(
        _agg_body,
        out_type=jax.ShapeDtypeStruct((NC, HALFP, D), jnp.float32),
        mesh=mesh,
        scratch_types=(
            [pltpu.VMEM((KCH, CH), jnp.int32),
             pltpu.VMEM((KCH, CH), jnp.int32)]
            + [pltpu.VMEM((CH, D), jnp.float32)] * NB
            + [pltpu.SemaphoreType.DMA] * (2 * NB)
            + [pltpu.VMEM_SHARED((HALFP, D), jnp.float32)]
        ),
    )
    return deg, agg


def _tc_a_body(hist_ref, x_ref, w_ref, dinv_ref, z_ref):
    counts = jnp.concatenate(
        [hist_ref[0, :HALF, :1], hist_ref[1, :HALF, :1]], axis=0)
    deg = counts + 1.0
    row = lax.broadcasted_iota(jnp.int32, (NPAD, 1), 0)
    dinv = jnp.where(row < N, lax.rsqrt(deg), 0.0)
    dinv_ref[...] = dinv
    z = jnp.dot(x_ref[...], w_ref[...], preferred_element_type=jnp.float32)
    z_ref[...] = z * dinv


_tc_a = pl.pallas_call(
    _tc_a_body,
    out_shape=(
        jax.ShapeDtypeStruct((NPAD, 1), jnp.float32),
        jax.ShapeDtypeStruct((NPAD, D), jnp.float32),
    ),
)


def _combine_bn_relu(p_ref, z_ref, dinv_ref, b_ref, g_ref, be_ref):
    dinv = dinv_ref[...]
    agg = jnp.concatenate([p_ref[0, :HALF], p_ref[1, :HALF]], axis=0)
    agg = agg + z_ref[...]
    pre = agg * dinv + b_ref[...]
    mask = lax.broadcasted_iota(jnp.int32, (NPAD, 1), 0) < N
    pre = jnp.where(mask, pre, 0.0)
    mean = jnp.sum(pre, axis=0, keepdims=True) * (1.0 / N)
    cent = jnp.where(mask, pre - mean, 0.0)
    var = jnp.sum(cent * cent, axis=0, keepdims=True) * (1.0 / N)
    h = g_ref[...] * cent * lax.rsqrt(var + EPS_BN) + be_ref[...]
    return jnp.maximum(h, 0.0)


def _tc_b_body(p_ref, z_ref, dinv_ref, b_ref, g_ref, be_ref, w_ref, z2_ref):
    h = _combine_bn_relu(p_ref, z_ref, dinv_ref, b_ref, g_ref, be_ref)
    z2 = jnp.dot(h, w_ref[...], preferred_element_type=jnp.float32)
    z2_ref[...] = z2 * dinv_ref[...]


_tc_b = pl.pallas_call(
    _tc_b_body,
    out_shape=jax.ShapeDtypeStruct((NPAD, D), jnp.float32),
)


def _tc_c_body(p_ref, z_ref, dinv_ref, b_ref, g_ref, be_ref, fair_ref,
               wl1a_ref, wl1b_ref, bl1_ref, wl2_ref, bl2_ref, out_ref):
    h = _combine_bn_relu(p_ref, z_ref, dinv_ref, b_ref, g_ref, be_ref)
    hl = jnp.dot(h, wl1a_ref[...], preferred_element_type=jnp.float32)
    hl = hl + jnp.dot(fair_ref[...], wl1b_ref[...],
                      preferred_element_type=jnp.float32)
    hl = jnp.maximum(hl + bl1_ref[...], 0.0)
    out = jnp.dot(hl, wl2_ref[...], preferred_element_type=jnp.float32)
    out_ref[...] = out + bl2_ref[...]


_tc_c = pl.pallas_call(
    _tc_c_body,
    out_shape=jax.ShapeDtypeStruct((NPAD, D_OUT), jnp.float32),
)


def kernel(x, adj_t, fair_node_embedding, W1, b1, g1, be1, W2, b2, g2, be2,
           Wl1, bl1, Wl2, bl2):
    src = adj_t[0].reshape(NS, EPT)
    dst = adj_t[1].reshape(NS, EPT)
    pad = jnp.full((NS, EPT_PAD - EPT), N, dtype=adj_t.dtype)
    src_p3 = jnp.concatenate([src, pad], axis=1).reshape(NS, KCH, CH)
    dst_p = jnp.concatenate([dst, pad], axis=1)
    spread = jax.lax.broadcasted_iota(jnp.int32, dst_p.shape, 1) % 64
    dloc = []
    for c in range(NC):
        d = dst_p - c * HALF
        dloc.append(jnp.where((d >= 0) & (d < HALF), d, TRASH + spread))
    dst_loc = jnp.stack(dloc).reshape(NC * NS, KCH, CH)

    x_pad = jnp.pad(x, ((0, NPAD - N), (0, 0)))
    fair_pad = jnp.pad(fair_node_embedding, ((0, NPAD - N), (0, 0)))
    zeros = jnp.zeros((HALFP, D), jnp.float32)
    ones = jnp.ones((CH, D), jnp.float32)

    deg_kernel, agg_kernel = _sc_kernels()
    hist = deg_kernel(dst_loc, ones, zeros)
    dinv, z1 = _tc_a(hist, x_pad, W1)
    p1 = agg_kernel(z1, src_p3, dst_loc, zeros)
    z2 = _tc_b(p1, z1, dinv, b1.reshape(1, D), g1.reshape(1, D),
               be1.reshape(1, D), W2)
    p2 = agg_kernel(z2, src_p3, dst_loc, zeros)
    out = _tc_c(p2, z2, dinv, b2.reshape(1, D), g2.reshape(1, D),
                be2.reshape(1, D), fair_pad, Wl1[:D], Wl1[D:],
                bl1.reshape(1, D), Wl2, bl2.reshape(1, D_OUT))
    return out[:N]


# speedup vs baseline: 1.0682x; 1.0003x over previous
import functools

import jax
import jax.numpy as jnp
from jax import lax
from jax.experimental import pallas as pl
from jax.experimental.pallas import tpu as pltpu
from jax.experimental.pallas import tpu_sc as plsc

N = 10000
E = 320000
D = 128
D_EMB = 64
D_OUT = 40
EPS_BN = 1e-5

NC = 2
NS = 16
NPAD = 10240
HALF = NPAD // NC
HALFP = 5248
TRASH = HALF
CH = 128
EPT = E // NS
KCH = 160
EPT_PAD = KCH * CH
RPT = HALFP // NS


DEGW = 4


def _deg_body(dst_hbm, ones_hbm, zeros_hbm, out_hbm, di_v, ones_v, semd,
              deg_sh):
    c = lax.axis_index("c")
    s = lax.axis_index("s")
    wid = c * NS + s
    row0 = s * RPT
    pltpu.sync_copy(dst_hbm.at[wid], di_v)
    pltpu.sync_copy(ones_hbm, ones_v)
    pltpu.sync_copy(zeros_hbm.at[pl.ds(row0, RPT)],
                    deg_sh.at[pl.ds(row0, RPT)])
    plsc.subcore_barrier()

    for b in range(DEGW):
        pltpu.async_copy(ones_v, deg_sh.at[di_v.at[b]], semd, add=True)

    def chunk_body(j, carry):
        pltpu.make_async_copy(ones_v, deg_sh.at[di_v.at[0]], semd).wait()

        @pl.when(j + DEGW < KCH)
        def _():
            pltpu.async_copy(ones_v, deg_sh.at[di_v.at[j + DEGW]], semd,
                             add=True)
        return carry

    lax.fori_loop(0, KCH, chunk_body, 0)
    plsc.subcore_barrier()
    pltpu.sync_copy(deg_sh.at[pl.ds(row0, RPT)],
                    out_hbm.at[c].at[pl.ds(row0, RPT)])


NB = 2


def _agg_body(z_hbm, src_hbm, dst_hbm, zeros_hbm, p_hbm,
              si_v, di_v, gbuf0, gbuf1,
              semg0, semg1, sems0, sems1, agg_sh):
    c = lax.axis_index("c")
    s = lax.axis_index("s")
    wid = c * NS + s
    row0 = s * RPT
    pltpu.sync_copy(src_hbm.at[s], si_v)
    pltpu.sync_copy(dst_hbm.at[wid], di_v)
    pltpu.sync_copy(zeros_hbm.at[pl.ds(row0, RPT)],
                    agg_sh.at[pl.ds(row0, RPT)])
    plsc.subcore_barrier()

    gbufs = (gbuf0, gbuf1)
    semg = (semg0, semg1)
    sems = (sems0, sems1)
    for b in range(NB):
        pltpu.async_copy(z_hbm.at[si_v.at[b]], gbufs[b], semg[b])

    def ring_body(i, carry):
        j0 = i * NB
        for b in range(NB):
            j = j0 + b
            pltpu.make_async_copy(z_hbm.at[si_v.at[0]], gbufs[b],
                                  semg[b]).wait()
            pltpu.async_copy(gbufs[b], agg_sh.at[di_v.at[j]], sems[b],
                             add=True)
        for b in range(NB):
            j = j0 + b

            @pl.when(j + NB < KCH)
            def _():
                pltpu.make_async_copy(gbufs[b], agg_sh.at[di_v.at[0]],
                                      sems[b]).wait()
                pltpu.async_copy(z_hbm.at[si_v.at[j + NB]], gbufs[b],
                                 semg[b])
        return carry

    lax.fori_loop(0, KCH // NB, ring_body, 0)
    for b in range(NB):
        pltpu.make_async_copy(gbufs[b], agg_sh.at[di_v.at[0]], sems[b]).wait()
    plsc.subcore_barrier()
    pltpu.sync_copy(agg_sh.at[pl.ds(row0, RPT)],
                    p_hbm.at[c].at[pl.ds(row0, RPT)])


@functools.cache
def _sc_kernels():
    mesh = plsc.VectorSubcoreMesh(core_axis_name="c", subcore_axis_name="s")
    deg = pl.kernel(
        _deg_body,
        out_type=jax.ShapeDtypeStruct((NC, HALFP, D), jnp.float32),
        mesh=mesh,
        scratch_types=[
            pltpu.VMEM((KCH, CH), jnp.int32),
            pltpu.VMEM((CH, D), jnp.float32),
            pltpu.SemaphoreType.DMA,
            pltpu.VMEM_SHARED((HALFP, D), jnp.float32),
        ],
    )
    agg = pl.kernel(
        _agg_body,
        out_type=jax.ShapeDtypeStruct((NC, HALFP, D), jnp.float32),
        mesh=mesh,
        scratch_types=(
            [pltpu.VMEM((KCH, CH), jnp.int32),
             pltpu.VMEM((KCH, CH), jnp.int32)]
            + [pltpu.VMEM((CH, D), jnp.float32)] * NB
            + [pltpu.SemaphoreType.DMA] * (2 * NB)
            + [pltpu.VMEM_SHARED((HALFP, D), jnp.float32)]
        ),
    )
    return deg, agg


def _tc_a_body(hist_ref, x_ref, w_ref, dinv_ref, z_ref):
    counts = jnp.concatenate(
        [hist_ref[0, :HALF, :1], hist_ref[1, :HALF, :1]], axis=0)
    deg = counts + 1.0
    row = lax.broadcasted_iota(jnp.int32, (NPAD, 1), 0)
    dinv = jnp.where(row < N, lax.rsqrt(deg), 0.0)
    dinv_ref[...] = dinv
    z = jnp.dot(x_ref[...], w_ref[...], preferred_element_type=jnp.float32)
    z_ref[...] = z * dinv


_tc_a = pl.pallas_call(
    _tc_a_body,
    out_shape=(
        jax.ShapeDtypeStruct((NPAD, 1), jnp.float32),
        jax.ShapeDtypeStruct((NPAD, D), jnp.float32),
    ),
)


def _combine_bn_relu(p_ref, z_ref, dinv_ref, b_ref, g_ref, be_ref):
    dinv = dinv_ref[...]
    agg = jnp.concatenate([p_ref[0, :HALF], p_ref[1, :HALF]], axis=0)
    agg = agg + z_ref[...]
    pre = agg * dinv + b_ref[...]
    mask = lax.broadcasted_iota(jnp.int32, (NPAD, 1), 0) < N
    pre = jnp.where(mask, pre, 0.0)
    mean = jnp.sum(pre, axis=0, keepdims=True) * (1.0 / N)
    cent = jnp.where(mask, pre - mean, 0.0)
    var = jnp.sum(cent * cent, axis=0, keepdims=True) * (1.0 / N)
    h = g_ref[...] * cent * lax.rsqrt(var + EPS_BN) + be_ref[...]
    return jnp.maximum(h, 0.0)


def _tc_b_body(p_ref, z_ref, dinv_ref, b_ref, g_ref, be_ref, w_ref, z2_ref):
    h = _combine_bn_relu(p_ref, z_ref, dinv_ref, b_ref, g_ref, be_ref)
    z2 = jnp.dot(h, w_ref[...], preferred_element_type=jnp.float32)
    z2_ref[...] = z2 * dinv_ref[...]


_tc_b = pl.pallas_call(
    _tc_b_body,
    out_shape=jax.ShapeDtypeStruct((NPAD, D), jnp.float32),
)


def _tc_c_body(p_ref, z_ref, dinv_ref, b_ref, g_ref, be_ref, fair_ref,
               wl1a_ref, wl1b_ref, bl1_ref, wl2_ref, bl2_ref, out_ref):
    h = _combine_bn_relu(p_ref, z_ref, dinv_ref, b_ref, g_ref, be_ref)
    hl = jnp.dot(h, wl1a_ref[...], preferred_element_type=jnp.float32)
    hl = hl + jnp.dot(fair_ref[...], wl1b_ref[...],
                      preferred_element_type=jnp.float32)
    hl = jnp.maximum(hl + bl1_ref[...], 0.0)
    out = jnp.dot(hl, wl2_ref[...], preferred_element_type=jnp.float32)
    out_ref[...] = out + bl2_ref[...]


_tc_c = pl.pallas_call(
    _tc_c_body,
    out_shape=jax.ShapeDtypeStruct((NPAD, D_OUT), jnp.float32),
)


def kernel(x, adj_t, fair_node_embedding, W1, b1, g1, be1, W2, b2, g2, be2,
           Wl1, bl1, Wl2, bl2):
    src = adj_t[0].reshape(NS, EPT)
    dst = adj_t[1].reshape(NS, EPT)
    pad = jnp.full((NS, EPT_PAD - EPT), N, dtype=adj_t.dtype)
    src_p3 = jnp.concatenate([src, pad], axis=1).reshape(NS, KCH, CH)
    dst_p = jnp.concatenate([dst, pad], axis=1)
    spread = jax.lax.broadcasted_iota(jnp.int32, dst_p.shape, 1) % 64
    dloc = []
    for c in range(NC):
        d = dst_p - c * HALF
        dloc.append(jnp.where((d >= 0) & (d < HALF), d, TRASH + spread))
    dst_loc = jnp.stack(dloc).reshape(NC * NS, KCH, CH)

    x_pad = jnp.pad(x, ((0, NPAD - N), (0, 0)))
    fair_pad = jnp.pad(fair_node_embedding, ((0, NPAD - N), (0, 0)))
    zeros = jnp.zeros((HALFP, D), jnp.float32)
    ones = jnp.ones((CH, D), jnp.float32)

    deg_kernel, agg_kernel = _sc_kernels()
    hist = deg_kernel(dst_loc, ones, zeros)
    dinv, z1 = _tc_a(hist, x_pad, W1)
    p1 = agg_kernel(z1, src_p3, dst_loc, zeros)
    z2 = _tc_b(p1, z1, dinv, b1.reshape(1, D), g1.reshape(1, D),
               be1.reshape(1, D), W2)
    p2 = agg_kernel(z2, src_p3, dst_loc, zeros)
    out = _tc_c(p2, z2, dinv, b2.reshape(1, D), g2.reshape(1, D),
                be2.reshape(1, D), fair_pad, Wl1[:D], Wl1[D:],
                bl1.reshape(1, D), Wl2, bl2.reshape(1, D_OUT))
    return out[:N]
